# baseline (speedup 1.0000x reference)
import functools

import jax
import jax.numpy as jnp
from jax import lax
from jax.experimental import pallas as pl
from jax.experimental.pallas import tpu as pltpu
from jax.experimental.pallas import tpu_sc as plsc

_N = 9870
_D = 128
_H = 128
_B = 141
_E = 157920

_NW = 32
_CHUNK = 128
_NCH = 40
_EPW = _CHUNK * _NCH
_STRIPE = 624
_ACC = 15 * _STRIPE + 512
_NPAD = 9984

_R = 1248
_NB = _NPAD // _R

_HI = lax.Precision.HIGHEST


def _dot(a, b):
    return jnp.dot(a, b, preferred_element_type=jnp.float32, precision=_HI)


def _dot_d(a, b):
    return jnp.dot(a, b, preferred_element_type=jnp.float32)


def _onehot(lo, hi, j):
    r = (j * _R + lax.broadcasted_iota(jnp.int32, (_R, _B), 0))
    return ((r >= lo) & (r < hi)).astype(jnp.float32)


def _softmax(al):
    m = jnp.max(al, axis=-1, keepdims=True)
    e = jnp.exp(al - m)
    return e / jnp.sum(e, axis=-1, keepdims=True)


def _gnew_of(out, g, glbW1, glbW2, glbb):
    return g + jnp.tanh(_dot(out, glbW1) + _dot(g, glbW2) + glbb)


def _relu_rows(agg_ref, q_ref, bl_ref):
    h = jnp.maximum(agg_ref[0] + agg_ref[1] + bl_ref[...] + q_ref[...], 0.0)
    r = (pl.program_id(0) * _R
         + lax.broadcasted_iota(jnp.int32, (_R, 1), 0))
    return jnp.where(r < _N, h, 0.0)


def _p1_body(first, last):
    def body(*refs):
        if first:
            (x_ref, lo_ref, hi_ref, attW, attb, featW, featb, out_ref) = refs
            h = x_ref[...]
        elif last:
            (agg_ref, q_ref, bl_ref, lo_ref, hi_ref, g_ref, attW, attb,
             featW, featb, glbW1, glbW2, glbb, h_out, out_ref, g_out) = refs
            h = _relu_rows(agg_ref, q_ref, bl_ref)
            h_out[...] = h
        else:
            (agg_ref, q_ref, bl_ref, lo_ref, hi_ref, attW, attb, featW,
             featb, h_out, out_ref) = refs
            h = _relu_rows(agg_ref, q_ref, bl_ref)
            h_out[...] = h
        j = pl.program_id(0)
        a = _softmax(_dot_d(h, attW[...]) + attb[...])
        f = _dot_d(h, featW[...]) + featb[...]
        prod = a * f
        oh = _onehot(lo_ref[...], hi_ref[...], j)
        contrib = lax.dot_general(oh, prod, (((0,), (0,)), ((), ())),
                                  preferred_element_type=jnp.float32)

        @pl.when(j == 0)
        def _():
            out_ref[...] = jnp.zeros_like(out_ref)

        out_ref[...] += contrib
        if last:
            @pl.when(j == _NB - 1)
            def _():
                g_out[...] = _gnew_of(out_ref[...], g_ref[...], glbW1[...],
                                      glbW2[...], glbb[...])
    return body


def _p3p_body(h_ref, lo_ref, hi_ref, out_ref, g_ref, glbW1, glbW2, glbb,
              wlt, wlb, g_out, p_out):
    j = pl.program_id(0)
    gnew = _gnew_of(out_ref[...], g_ref[...], glbW1[...], glbW2[...],
                    glbb[...])

    @pl.when(j == 0)
    def _():
        g_out[...] = gnew

    oh = _onehot(lo_ref[...], hi_ref[...], j)
    p_out[...] = _dot(h_ref[...], wlt[...]) + _dot_d(oh, _dot(gnew, wlb[...]))


def _p3q_body(h_ref, lo_ref, hi_ref, g_ref, wrt, wrb, q_out):
    j = pl.program_id(0)
    oh = _onehot(lo_ref[...], hi_ref[...], j)
    q_out[...] = (_dot(h_ref[...], wrt[...])
                  + _dot_d(oh, _dot(g_ref[...], wrb[...])))


_row = pl.BlockSpec((_R, _H), lambda j: (j, 0))
_full = lambda r, c: pl.BlockSpec((r, c), lambda j: (0, 0))
_segb = pl.BlockSpec((1, _B), lambda j: (0, 0))
_aggb = pl.BlockSpec((2, _R, _H), lambda j: (0, j, 0))
_w = _full(_H, _H)
_b1 = _full(1, _H)
_gB = _full(_B, _H)

_f32 = jnp.float32


def _p1_first():
    return pl.pallas_call(
        _p1_body(True, False),
        grid=(_NB,),
        in_specs=[_row, _segb, _segb, _w, _b1, _w, _b1],
        out_specs=_gB,
        out_shape=jax.ShapeDtypeStruct((_B, _H), _f32),
    )


def _p1_mid():
    return pl.pallas_call(
        _p1_body(False, False),
        grid=(_NB,),
        in_specs=[_aggb, _row, _b1, _segb, _segb, _w, _b1, _w, _b1],
        out_specs=(_row, _gB),
        out_shape=(jax.ShapeDtypeStruct((_NPAD, _H), _f32),
                   jax.ShapeDtypeStruct((_B, _H), _f32)),
    )


def _p1_last():
    return pl.pallas_call(
        _p1_body(False, True),
        grid=(_NB,),
        in_specs=[_aggb, _row, _b1, _segb, _segb, _gB, _w, _b1, _w, _b1,
                  _w, _w, _b1],
        out_specs=(_row, _gB, _gB),
        out_shape=(jax.ShapeDtypeStruct((_NPAD, _H), _f32),
                   jax.ShapeDtypeStruct((_B, _H), _f32),
                   jax.ShapeDtypeStruct((_B, _H), _f32)),
    )


def _p3p():
    return pl.pallas_call(
        _p3p_body,
        grid=(_NB,),
        in_specs=[_row, _segb, _segb, _gB, _gB, _w, _w, _b1, _w, _w],
        out_specs=(_gB, _row),
        out_shape=(jax.ShapeDtypeStruct((_B, _H), _f32),
                   jax.ShapeDtypeStruct((_NPAD, _H), _f32)),
    )


def _p3q():
    return pl.pallas_call(
        _p3q_body,
        grid=(_NB,),
        in_specs=[_row, _segb, _segb, _gB, _w, _w],
        out_specs=_row,
        out_shape=jax.ShapeDtypeStruct((_NPAD, _H), _f32),
    )


def _sc_segment_sum(p, src_p, dst_p, zer):
    @functools.partial(
        pl.kernel,
        out_type=jax.ShapeDtypeStruct((2, _ACC, _H), jnp.float32),
        mesh=plsc.VectorSubcoreMesh(core_axis_name="c", subcore_axis_name="s"),
        scratch_types=[
            pltpu.VMEM((_NCH, _CHUNK), jnp.int32),
            pltpu.VMEM((_NCH, _CHUNK), jnp.int32),
            pltpu.VMEM((2, _CHUNK, _H), jnp.float32),
            pltpu.VMEM_SHARED((_ACC, _H), jnp.float32),
            pltpu.SemaphoreType.DMA,
            pltpu.SemaphoreType.DMA,
            pltpu.SemaphoreType.DMA,
            pltpu.SemaphoreType.DMA,
        ],
    )
    def k(p_hbm, src_hbm, dst_hbm, zer_hbm, out_hbm, src_v, dst_v, rows,
          acc_sh, sg0, sg1, ss0, ss1):
        sem_g = [sg0, sg1]
        sem_s = [ss0, ss1]
        c = lax.axis_index("c")
        s = lax.axis_index("s")
        wid = s * 2 + c
        @pl.when(s < 15)
        def _():
            pltpu.sync_copy(zer_hbm, acc_sh.at[pl.ds(s * _STRIPE, _STRIPE)])

        @pl.when(s == 15)
        def _():
            pltpu.sync_copy(zer_hbm.at[pl.ds(0, 512)],
                            acc_sh.at[pl.ds(15 * _STRIPE, 512)])

        pltpu.sync_copy(src_hbm.at[wid], src_v)
        pltpu.sync_copy(dst_hbm.at[wid], dst_v)
        plsc.subcore_barrier()

        for k2 in range(2):
            pltpu.async_copy(p_hbm.at[src_v.at[k2]], rows.at[k2],
                             sem_g[k2])

        def body(t, carry):
            j0 = 2 * t
            for k2 in range(2):
                pltpu.make_async_copy(p_hbm.at[src_v.at[j0 + k2]],
                                      rows.at[k2], sem_g[k2]).wait()
                pltpu.async_copy(rows.at[k2], acc_sh.at[dst_v.at[j0 + k2]],
                                 sem_s[k2], add=True)
            for k2 in range(2):
                @pl.when(j0 + 2 + k2 < _NCH)
                def _():
                    pltpu.make_async_copy(
                        rows.at[k2], acc_sh.at[dst_v.at[j0 + k2]],
                        sem_s[k2]).wait()
                    pltpu.async_copy(p_hbm.at[src_v.at[j0 + 2 + k2]],
                                     rows.at[k2], sem_g[k2])
            return carry

        lax.fori_loop(0, _NCH // 2, body, 0)
        for k2 in range(2):
            pltpu.make_async_copy(rows.at[k2],
                                  acc_sh.at[dst_v.at[_NCH - 2 + k2]],
                                  sem_s[k2]).wait()
        plsc.subcore_barrier()

        @pl.when(s < 15)
        def _():
            pltpu.sync_copy(acc_sh.at[pl.ds(s * _STRIPE, _STRIPE)],
                            out_hbm.at[c].at[pl.ds(s * _STRIPE, _STRIPE)])

        @pl.when(s == 15)
        def _():
            pltpu.sync_copy(acc_sh.at[pl.ds(15 * _STRIPE, 512)],
                            out_hbm.at[c].at[pl.ds(15 * _STRIPE, 512)])

    return k(p, src_p, dst_p, zer)


def kernel(x, edge_index, batch_sizes, att_W, att_b, feat_W, feat_b,
           glb_W, glb_b, sage_Wl, sage_bl, sage_Wr):
    offs = jnp.concatenate([jnp.zeros((1,), jnp.int32),
                            jnp.cumsum(batch_sizes, dtype=jnp.int32)])
    lo = offs[:_B].reshape(1, _B)
    hi = offs[1:].reshape(1, _B)
    x_pad = jnp.concatenate(
        [x, jnp.zeros((_NPAD - _N, _D), jnp.float32)], axis=0)
    src = edge_index[0]
    dst = edge_index[1]
    pad = _NW * _EPW - _E
    fidx = jnp.arange(pad, dtype=jnp.int32)
    src_p = jnp.concatenate(
        [src, fidx % _N]).reshape(_NW, _NCH, _CHUNK)
    dst_p = jnp.concatenate(
        [dst, _N + fidx % (_ACC - _N)]).reshape(_NW, _NCH, _CHUNK)
    zer = jnp.zeros((_STRIPE, _H), jnp.float32)
    g = jnp.zeros((_B, _H), jnp.float32)

    aw = lambda i: att_W[i]
    ab = lambda i: att_b[i].reshape(1, _H)
    fw = lambda i: feat_W[i]
    fb = lambda i: feat_b[i].reshape(1, _H)
    gw1 = lambda i: glb_W[i, :_H]
    gw2 = lambda i: glb_W[i, _H:]
    gb = lambda i: glb_b[i].reshape(1, _H)
    wlt = lambda i: sage_Wl[i, :_D]
    wlb = lambda i: sage_Wl[i, _D:]
    wrt = lambda i: sage_Wr[i, :_D]
    wrb = lambda i: sage_Wr[i, _D:]
    blr = lambda i: sage_bl[i].reshape(1, _H)

    out0 = _p1_first()(x_pad, lo, hi, aw(0), ab(0), fw(0), fb(0))
    g, p = _p3p()(x_pad, lo, hi, out0, g, gw1(0), gw2(0), gb(0),
                  wlt(0), wlb(0))
    q = _p3q()(x_pad, lo, hi, g, wrt(0), wrb(0))
    for i in (1, 2):
        aggs = _sc_segment_sum(p, src_p, dst_p, zer)
        h, out = _p1_mid()(aggs, q, blr(i - 1), lo, hi, aw(i), ab(i),
                           fw(i), fb(i))
        g, p = _p3p()(h, lo, hi, out, g, gw1(i), gw2(i), gb(i),
                      wlt(i), wlb(i))
        q = _p3q()(h, lo, hi, g, wrt(i), wrb(i))
    aggs = _sc_segment_sum(p, src_p, dst_p, zer)
    h, _, g = _p1_last()(aggs, q, blr(2), lo, hi, g, aw(3), ab(3), fw(3),
                         fb(3), gw1(3), gw2(3), gb(3))
    return (h[:_N], g)


# speedup vs baseline: 1.1628x; 1.1628x over previous
import functools

import jax
import jax.numpy as jnp
from jax import lax
from jax.experimental import pallas as pl
from jax.experimental.pallas import tpu as pltpu
from jax.experimental.pallas import tpu_sc as plsc

_N = 9870
_D = 128
_H = 128
_B = 141
_E = 157920

_NW = 32
_CHUNK = 128
_NCH = 40
_EPW = _CHUNK * _NCH
_STRIPE = 624
_ACC = 15 * _STRIPE + 512
_NPAD = 9984

_R = 1248
_NB = _NPAD // _R

_HI = lax.Precision.HIGHEST


def _dot(a, b):
    return jnp.dot(a, b, preferred_element_type=jnp.float32, precision=_HI)


def _dot_d(a, b):
    return jnp.dot(a, b, preferred_element_type=jnp.float32)


def _onehot(lo, hi, j):
    r = (j * _R + lax.broadcasted_iota(jnp.int32, (_R, _B), 0))
    return ((r >= lo) & (r < hi)).astype(jnp.float32)


def _softmax(al):
    m = jnp.max(al, axis=-1, keepdims=True)
    e = jnp.exp(al - m)
    return e / jnp.sum(e, axis=-1, keepdims=True)


def _gnew_of(out, g, glbW1, glbW2, glbb):
    return g + jnp.tanh(_dot(out, glbW1) + _dot(g, glbW2) + glbb)


def _relu_rows(agg_ref, q_ref, bl_ref):
    h = jnp.maximum(agg_ref[0] + agg_ref[1] + bl_ref[...] + q_ref[...], 0.0)
    r = (pl.program_id(0) * _R
         + lax.broadcasted_iota(jnp.int32, (_R, 1), 0))
    return jnp.where(r < _N, h, 0.0)


def _p1_body(first, last):
    def body(*refs):
        if first:
            (x_ref, lo_ref, hi_ref, attW, attb, featW, featb, out_ref) = refs
            h = x_ref[...]
        elif last:
            (agg_ref, q_ref, bl_ref, lo_ref, hi_ref, g_ref, attW, attb,
             featW, featb, glbW1, glbW2, glbb, h_out, out_ref, g_out) = refs
            h = _relu_rows(agg_ref, q_ref, bl_ref)
            h_out[...] = h
        else:
            (agg_ref, q_ref, bl_ref, lo_ref, hi_ref, attW, attb, featW,
             featb, h_out, out_ref) = refs
            h = _relu_rows(agg_ref, q_ref, bl_ref)
            h_out[...] = h
        j = pl.program_id(0)
        a = _softmax(_dot_d(h, attW[...]) + attb[...])
        f = _dot_d(h, featW[...]) + featb[...]
        prod = a * f
        oh = _onehot(lo_ref[...], hi_ref[...], j)
        contrib = lax.dot_general(oh, prod, (((0,), (0,)), ((), ())),
                                  preferred_element_type=jnp.float32)

        @pl.when(j == 0)
        def _():
            out_ref[...] = jnp.zeros_like(out_ref)

        out_ref[...] += contrib
        if last:
            @pl.when(j == _NB - 1)
            def _():
                g_out[...] = _gnew_of(out_ref[...], g_ref[...], glbW1[...],
                                      glbW2[...], glbb[...])
    return body


def _p3p_body(h_ref, lo_ref, hi_ref, out_ref, g_ref, glbW1, glbW2, glbb,
              wlt, wlb, g_out, p_out):
    j = pl.program_id(0)
    gnew = _gnew_of(out_ref[...], g_ref[...], glbW1[...], glbW2[...],
                    glbb[...])

    @pl.when(j == 0)
    def _():
        g_out[...] = gnew

    oh = _onehot(lo_ref[...], hi_ref[...], j)
    p_out[...] = _dot(h_ref[...], wlt[...]) + _dot_d(oh, _dot(gnew, wlb[...]))


def _p3q_body(h_ref, lo_ref, hi_ref, g_ref, wrt, wrb, q_out):
    j = pl.program_id(0)
    oh = _onehot(lo_ref[...], hi_ref[...], j)
    q_out[...] = (_dot(h_ref[...], wrt[...])
                  + _dot_d(oh, _dot(g_ref[...], wrb[...])))


_row = pl.BlockSpec((_R, _H), lambda j: (j, 0))
_full = lambda r, c: pl.BlockSpec((r, c), lambda j: (0, 0))
_segb = pl.BlockSpec((1, _B), lambda j: (0, 0))
_aggb = pl.BlockSpec((2, _R, _H), lambda j: (0, j, 0))
_w = _full(_H, _H)
_b1 = _full(1, _H)
_gB = _full(_B, _H)

_f32 = jnp.float32


def _p1_first():
    return pl.pallas_call(
        _p1_body(True, False),
        grid=(_NB,),
        in_specs=[_row, _segb, _segb, _w, _b1, _w, _b1],
        out_specs=_gB,
        out_shape=jax.ShapeDtypeStruct((_B, _H), _f32),
    )


def _p1_mid():
    return pl.pallas_call(
        _p1_body(False, False),
        grid=(_NB,),
        in_specs=[_aggb, _row, _b1, _segb, _segb, _w, _b1, _w, _b1],
        out_specs=(_row, _gB),
        out_shape=(jax.ShapeDtypeStruct((_NPAD, _H), _f32),
                   jax.ShapeDtypeStruct((_B, _H), _f32)),
    )


def _p1_last():
    return pl.pallas_call(
        _p1_body(False, True),
        grid=(_NB,),
        in_specs=[_aggb, _row, _b1, _segb, _segb, _gB, _w, _b1, _w, _b1,
                  _w, _w, _b1],
        out_specs=(_row, _gB, _gB),
        out_shape=(jax.ShapeDtypeStruct((_NPAD, _H), _f32),
                   jax.ShapeDtypeStruct((_B, _H), _f32),
                   jax.ShapeDtypeStruct((_B, _H), _f32)),
    )


def _p3p():
    return pl.pallas_call(
        _p3p_body,
        grid=(_NB,),
        in_specs=[_row, _segb, _segb, _gB, _gB, _w, _w, _b1, _w, _w],
        out_specs=(_gB, _row),
        out_shape=(jax.ShapeDtypeStruct((_B, _H), _f32),
                   jax.ShapeDtypeStruct((_NPAD, _H), _f32)),
    )


def _p3q():
    return pl.pallas_call(
        _p3q_body,
        grid=(_NB,),
        in_specs=[_row, _segb, _segb, _gB, _w, _w],
        out_specs=_row,
        out_shape=jax.ShapeDtypeStruct((_NPAD, _H), _f32),
    )


def _sc_segment_sum(p, src_p, dst_p, zer):
    @functools.partial(
        pl.kernel,
        out_type=jax.ShapeDtypeStruct((2, _ACC, _H), jnp.float32),
        mesh=plsc.VectorSubcoreMesh(core_axis_name="c", subcore_axis_name="s"),
        scratch_types=[
            pltpu.VMEM((_NCH, _CHUNK), jnp.int32),
            pltpu.VMEM((_NCH, _CHUNK), jnp.int32),
            pltpu.VMEM((2, _CHUNK, _H), jnp.float32),
            pltpu.VMEM_SHARED((_ACC, _H), jnp.float32),
            pltpu.SemaphoreType.DMA,
            pltpu.SemaphoreType.DMA,
        ],
    )
    def k(p_hbm, src_hbm, dst_hbm, zer_hbm, out_hbm, src_v, dst_v, rows,
          acc_sh, sg0, sg1):
        sem_g = [sg0, sg1]
        c = lax.axis_index("c")
        s = lax.axis_index("s")
        wid = s * 2 + c
        @pl.when(s < 15)
        def _():
            pltpu.sync_copy(zer_hbm, acc_sh.at[pl.ds(s * _STRIPE, _STRIPE)])

        @pl.when(s == 15)
        def _():
            pltpu.sync_copy(zer_hbm.at[pl.ds(0, 512)],
                            acc_sh.at[pl.ds(15 * _STRIPE, 512)])

        pltpu.sync_copy(src_hbm.at[wid], src_v)
        pltpu.sync_copy(dst_hbm.at[wid], dst_v)
        plsc.subcore_barrier()

        pltpu.async_copy(p_hbm.at[src_v.at[0]], rows.at[0], sem_g[0])
        pltpu.async_copy(p_hbm.at[src_v.at[1]], rows.at[1], sem_g[1])

        def body(t, carry):
            j0 = 2 * t
            for k2 in range(2):
                pltpu.make_async_copy(p_hbm.at[src_v.at[j0 + k2]],
                                      rows.at[k2], sem_g[k2]).wait()
                pltpu.sync_copy(rows.at[k2], acc_sh.at[dst_v.at[j0 + k2]],
                                add=True)

                @pl.when(j0 + 2 + k2 < _NCH)
                def _():
                    pltpu.async_copy(p_hbm.at[src_v.at[j0 + 2 + k2]],
                                     rows.at[k2], sem_g[k2])
            return carry

        lax.fori_loop(0, _NCH // 2, body, 0)
        plsc.subcore_barrier()

        @pl.when(s < 15)
        def _():
            pltpu.sync_copy(acc_sh.at[pl.ds(s * _STRIPE, _STRIPE)],
                            out_hbm.at[c].at[pl.ds(s * _STRIPE, _STRIPE)])

        @pl.when(s == 15)
        def _():
            pltpu.sync_copy(acc_sh.at[pl.ds(15 * _STRIPE, 512)],
                            out_hbm.at[c].at[pl.ds(15 * _STRIPE, 512)])

    return k(p, src_p, dst_p, zer)


def kernel(x, edge_index, batch_sizes, att_W, att_b, feat_W, feat_b,
           glb_W, glb_b, sage_Wl, sage_bl, sage_Wr):
    offs = jnp.concatenate([jnp.zeros((1,), jnp.int32),
                            jnp.cumsum(batch_sizes, dtype=jnp.int32)])
    lo = offs[:_B].reshape(1, _B)
    hi = offs[1:].reshape(1, _B)
    x_pad = jnp.concatenate(
        [x, jnp.zeros((_NPAD - _N, _D), jnp.float32)], axis=0)
    src = edge_index[0]
    dst = edge_index[1]
    pad = _NW * _EPW - _E
    fidx = jnp.arange(pad, dtype=jnp.int32)
    src_p = jnp.concatenate(
        [src, fidx % _N]).reshape(_NW, _NCH, _CHUNK)
    dst_p = jnp.concatenate(
        [dst, _N + fidx % (_ACC - _N)]).reshape(_NW, _NCH, _CHUNK)
    zer = jnp.zeros((_STRIPE, _H), jnp.float32)
    g = jnp.zeros((_B, _H), jnp.float32)

    aw = lambda i: att_W[i]
    ab = lambda i: att_b[i].reshape(1, _H)
    fw = lambda i: feat_W[i]
    fb = lambda i: feat_b[i].reshape(1, _H)
    gw1 = lambda i: glb_W[i, :_H]
    gw2 = lambda i: glb_W[i, _H:]
    gb = lambda i: glb_b[i].reshape(1, _H)
    wlt = lambda i: sage_Wl[i, :_D]
    wlb = lambda i: sage_Wl[i, _D:]
    wrt = lambda i: sage_Wr[i, :_D]
    wrb = lambda i: sage_Wr[i, _D:]
    blr = lambda i: sage_bl[i].reshape(1, _H)

    out0 = _p1_first()(x_pad, lo, hi, aw(0), ab(0), fw(0), fb(0))
    g, p = _p3p()(x_pad, lo, hi, out0, g, gw1(0), gw2(0), gb(0),
                  wlt(0), wlb(0))
    q = _p3q()(x_pad, lo, hi, g, wrt(0), wrb(0))
    for i in (1, 2):
        aggs = _sc_segment_sum(p, src_p, dst_p, zer)
        h, out = _p1_mid()(aggs, q, blr(i - 1), lo, hi, aw(i), ab(i),
                           fw(i), fb(i))
        g, p = _p3p()(h, lo, hi, out, g, gw1(i), gw2(i), gb(i),
                      wlt(i), wlb(i))
        q = _p3q()(h, lo, hi, g, wrt(i), wrb(i))
    aggs = _sc_segment_sum(p, src_p, dst_p, zer)
    h, _, g = _p1_last()(aggs, q, blr(2), lo, hi, g, aw(3), ab(3), fw(3),
                         fb(3), gw1(3), gw2(3), gb(3))
    return (h[:_N], g)
